# baseline (speedup 1.0000x reference)
import functools

import jax
import jax.numpy as jnp
from jax import lax
from jax.experimental import pallas as pl
from jax.experimental.pallas import tpu as pltpu
from jax.experimental.pallas import tpu_sc as plsc

VOCAB = 1000000
MAXLEN = 200
EMBED = 64
BATCH = 1024

N_ROWS = BATCH * MAXLEN
NW = 32
ROWS_PER_W = N_ROWS // NW
CHUNK = 100
NCHUNK = ROWS_PER_W // CHUNK
LANES = 16


def _add_pos(buf, pos_v, off):
    def rbody(r, carry):
        for j in range(EMBED // LANES):
            sl = pl.ds(j * LANES, LANES)
            buf[r, sl] = buf[r, sl] + pos_v[off + r, sl]
        return carry
    lax.fori_loop(0, CHUNK, rbody, 0, unroll=2)


def _make_sc_kernel():
    mesh = plsc.VectorSubcoreMesh(core_axis_name="c", subcore_axis_name="s")

    @functools.partial(
        pl.kernel,
        mesh=mesh,
        out_type=jax.ShapeDtypeStruct((N_ROWS // CHUNK, CHUNK, EMBED),
                                      jnp.float32),
        scratch_types=[
            pltpu.VMEM((NCHUNK, CHUNK), jnp.int32),
            pltpu.VMEM((MAXLEN, EMBED), jnp.float32),
            pltpu.VMEM((CHUNK, EMBED), jnp.float32),
            pltpu.VMEM((CHUNK, EMBED), jnp.float32),
            pltpu.SemaphoreType.DMA,
            pltpu.SemaphoreType.DMA,
        ],
        compiler_params=pltpu.CompilerParams(use_tc_tiling_on_sc=False),
    )
    def k(x_hbm, tok_hbm, pos_hbm, out_hbm, idx_v, pos_v, rows0, rows1,
          sem0, sem1):
        wid = lax.axis_index("s") * 2 + lax.axis_index("c")
        pltpu.sync_copy(x_hbm.at[wid], idx_v)
        pltpu.sync_copy(pos_hbm, pos_v)
        base_chunk = wid * NCHUNK

        pltpu.async_copy(tok_hbm.at[idx_v.at[0]], rows0, sem0)

        def gbody(g, carry):
            c0 = 2 * g
            pltpu.make_async_copy(tok_hbm.at[idx_v.at[c0]], rows0, sem0).wait()
            pltpu.async_copy(tok_hbm.at[idx_v.at[c0 + 1]], rows1, sem1)
            _add_pos(rows0, pos_v, 0)
            pltpu.sync_copy(rows0, out_hbm.at[base_chunk + c0])

            pltpu.make_async_copy(tok_hbm.at[idx_v.at[c0 + 1]], rows1, sem1).wait()

            @pl.when(g < NCHUNK // 2 - 1)
            def _():
                pltpu.async_copy(tok_hbm.at[idx_v.at[c0 + 2]], rows0, sem0)

            _add_pos(rows1, pos_v, CHUNK)
            pltpu.sync_copy(rows1, out_hbm.at[base_chunk + c0 + 1])
            return carry

        lax.fori_loop(0, NCHUNK // 2, gbody, 0)

    return k


_sc_kernel = _make_sc_kernel()


def kernel(x, token_table, pos_table):
    xi = x.astype(jnp.int32).reshape(NW, NCHUNK, CHUNK)
    out = _sc_kernel(xi, token_table, pos_table)
    return out.reshape(BATCH, MAXLEN, EMBED)


# speedup vs baseline: 1.0004x; 1.0004x over previous
import functools

import jax
import jax.numpy as jnp
from jax import lax
from jax.experimental import pallas as pl
from jax.experimental.pallas import tpu as pltpu
from jax.experimental.pallas import tpu_sc as plsc

VOCAB = 1000000
MAXLEN = 200
EMBED = 64
BATCH = 1024

N_ROWS = BATCH * MAXLEN
NW = 32
ROWS_PER_W = N_ROWS // NW
CHUNK = 128
NCHUNK = ROWS_PER_W // CHUNK
LANES = 16


def _add_pos(buf, pos_v, off):
    def rbody(r, carry):
        p = off + r
        p = jnp.where(p >= MAXLEN, p - MAXLEN, p)
        base = p * EMBED
        for j in range(EMBED // LANES):
            sl = pl.ds(j * LANES, LANES)
            psl = pl.ds(base + j * LANES, LANES)
            buf[r, sl] = buf[r, sl] + pos_v[psl]
        return carry
    lax.fori_loop(0, CHUNK, rbody, 0, unroll=4)


def _make_sc_kernel():
    mesh = plsc.VectorSubcoreMesh(core_axis_name="c", subcore_axis_name="s")

    @functools.partial(
        pl.kernel,
        mesh=mesh,
        out_type=jax.ShapeDtypeStruct((N_ROWS // CHUNK, CHUNK, EMBED),
                                      jnp.float32),
        scratch_types=[
            pltpu.VMEM((ROWS_PER_W,), jnp.int32),
            pltpu.VMEM((MAXLEN * EMBED,), jnp.float32),
            pltpu.VMEM((CHUNK, EMBED), jnp.float32),
            pltpu.VMEM((CHUNK, EMBED), jnp.float32),
            pltpu.SemaphoreType.DMA,
            pltpu.SemaphoreType.DMA,
        ],
        compiler_params=pltpu.CompilerParams(use_tc_tiling_on_sc=False),
    )
    def k(x_hbm, tok_hbm, pos_hbm, out_hbm, idx_v, pos_v, rows0, rows1,
          sem0, sem1):
        wid = lax.axis_index("s") * 2 + lax.axis_index("c")
        pltpu.sync_copy(x_hbm.at[pl.ds(wid * ROWS_PER_W, ROWS_PER_W)], idx_v)
        pltpu.sync_copy(pos_hbm, pos_v)
        base_chunk = wid * NCHUNK

        def idx_at(c):
            return idx_v.at[pl.ds(c * CHUNK, CHUNK)]

        pltpu.async_copy(tok_hbm.at[idx_at(0)], rows0, sem0)

        def gbody(g, carry):
            c0 = 2 * g
            pltpu.make_async_copy(tok_hbm.at[idx_at(c0)], rows0, sem0).wait()
            pltpu.async_copy(tok_hbm.at[idx_at(c0 + 1)], rows1, sem1)
            _add_pos(rows0, pos_v, lax.rem(c0 * CHUNK, MAXLEN))
            pltpu.sync_copy(rows0, out_hbm.at[base_chunk + c0])

            pltpu.make_async_copy(tok_hbm.at[idx_at(c0 + 1)], rows1, sem1).wait()

            @pl.when(g < NCHUNK // 2 - 1)
            def _():
                pltpu.async_copy(tok_hbm.at[idx_at(c0 + 2)], rows0, sem0)

            _add_pos(rows1, pos_v, lax.rem((c0 + 1) * CHUNK, MAXLEN))
            pltpu.sync_copy(rows1, out_hbm.at[base_chunk + c0 + 1])
            return carry

        lax.fori_loop(0, NCHUNK // 2, gbody, 0)

    return k


_sc_kernel = _make_sc_kernel()


def kernel(x, token_table, pos_table):
    x_flat = x.astype(jnp.int32).reshape(N_ROWS)
    pos_flat = pos_table.reshape(MAXLEN * EMBED)
    out = _sc_kernel(x_flat, token_table, pos_flat)
    return out.reshape(BATCH, MAXLEN, EMBED)
